# Initial kernel scaffold; baseline (speedup 1.0000x reference)
import jax
import jax.numpy as jnp
from jax.experimental import pallas as pl


def kernel(features, adj, W):
    raise NotImplementedError("write your pallas kernel here")


# speedup vs baseline: 1.8470x; 1.8470x over previous
import jax
import jax.numpy as jnp
from jax.experimental import pallas as pl
from jax.experimental.pallas import tpu as pltpu

_M = 400


def _proj_body(feat_ref, wt_ref, p1_ref, p2_ref):
    f = feat_ref[...]
    wt = wt_ref[...]
    d_in = f.shape[1]
    p1_ref[...] = jnp.dot(f, wt[:d_in, :], preferred_element_type=jnp.float32)
    p2_ref[...] = jnp.dot(f, wt[d_in:, :], preferred_element_type=jnp.float32)


def _main_body(p1_ref, adj_ref, p2_ref, out_ref):
    a = adj_ref[...]
    acc = jnp.dot(a, p2_ref[...], preferred_element_type=jnp.float32)
    denom = jnp.sum(a, axis=1, keepdims=True) + 1.0
    out_ref[...] = jnp.maximum(p1_ref[...] + acc / denom, 0.0)


def kernel(features, adj, W):
    n, d_in = features.shape
    d_out = W.shape[0]
    wt = W.T

    p1, p2 = pl.pallas_call(
        _proj_body,
        out_shape=(
            jax.ShapeDtypeStruct((n, d_out), jnp.float32),
            jax.ShapeDtypeStruct((n, d_out), jnp.float32),
        ),
    )(features, wt)

    out = pl.pallas_call(
        _main_body,
        grid=(n // _M,),
        in_specs=[
            pl.BlockSpec((_M, d_out), lambda i: (i, 0)),
            pl.BlockSpec((_M, n), lambda i: (i, 0)),
            pl.BlockSpec((n, d_out), lambda i: (0, 0)),
        ],
        out_specs=pl.BlockSpec((_M, d_out), lambda i: (i, 0)),
        out_shape=jax.ShapeDtypeStruct((n, d_out), jnp.float32),
        compiler_params=pltpu.CompilerParams(
            dimension_semantics=("parallel",),
        ),
    )(p1, adj, p2)
    return out
